# baseline (speedup 1.0000x reference)
import functools

import jax
import jax.numpy as jnp
from jax import lax
from jax.experimental import pallas as pl
from jax.experimental.pallas import tpu as pltpu
from jax.experimental.pallas import tpu_sc as plsc

B, AT, NBR, F = 8, 96, 16, 128
K = NBR - 1
NT = B * AT * NBR
NOUT = NT * K
NW = 32
ROWS_W = NOUT // NW
ATOMS_W = ROWS_W // (K * NBR)
CHUNK = 144
NCH = ROWS_W // CHUNK


@functools.partial(
    pl.kernel,
    mesh=plsc.VectorSubcoreMesh(core_axis_name="c", subcore_axis_name="s"),
    out_type=jax.ShapeDtypeStruct((NOUT, F), jnp.float32),
    compiler_params=pltpu.CompilerParams(needs_layout_passes=False),
    scratch_types=[
        pltpu.VMEM_SHARED((NT // 2, F), jnp.float32),
        pltpu.VMEM((ATOMS_W * NBR,), jnp.int32),
        pltpu.VMEM((ROWS_W,), jnp.int32),
        pltpu.VMEM((CHUNK, F), jnp.float32),
        pltpu.VMEM((CHUNK, F), jnp.float32),
        pltpu.VMEM((CHUNK, F), jnp.float32),
        pltpu.VMEM((CHUNK, F), jnp.float32),
        pltpu.SemaphoreType.DMA,
        pltpu.SemaphoreType.DMA,
        pltpu.SemaphoreType.DMA,
        pltpu.SemaphoreType.DMA,
        pltpu.SemaphoreType.DMA,
        pltpu.SemaphoreType.DMA,
        pltpu.SemaphoreType.DMA,
        pltpu.SemaphoreType.DMA,
        pltpu.SemaphoreType.DMA,
    ],
)
def _gather_kernel(table_hbm, nbr_hbm, out_hbm, slab, nbr_v, idx_v,
                   stage_0, stage_1, stage_2, stage_3,
                   gsem_0, gsem_1, gsem_2, gsem_3,
                   wsem_0, wsem_1, wsem_2, wsem_3, ssem):
    stages = [stage_0, stage_1, stage_2, stage_3]
    gsems = [gsem_0, gsem_1, gsem_2, gsem_3]
    wsems = [wsem_0, wsem_1, wsem_2, wsem_3]
    cid = lax.axis_index("c")
    sid = lax.axis_index("s")
    wid = cid * 16 + sid
    atom_base = wid * ATOMS_W
    row_base = wid * ROWS_W
    mol = atom_base // AT

    slab_rows = NT // 2 // 16
    slab_src = table_hbm.at[pl.ds(cid * (NT // 2) + sid * slab_rows, slab_rows)]
    slab_dst = slab.at[pl.ds(sid * slab_rows, slab_rows)]
    pltpu.async_copy(slab_src, slab_dst, ssem)
    pltpu.sync_copy(nbr_hbm.at[pl.ds(atom_base * NBR, ATOMS_W * NBR)], nbr_v)

    iota = lax.iota(jnp.int32, 16)
    kcols = [s - lax.shift_right_arithmetic(iota - (s + 1), 31) for s in range(K)]

    mol_loc = mol - cid * (B // 2)

    def build_atom(m, carry):
        base_vec = (nbr_v[pl.ds(m * NBR, NBR)] + mol_loc * AT) * NBR
        for s in range(K):
            idx_v[pl.ds((m * K + s) * NBR, NBR)] = base_vec + kcols[s]
        return carry

    lax.fori_loop(0, ATOMS_W, build_atom, 0)

    pltpu.make_async_copy(slab_src, slab_dst, ssem).wait()
    plsc.subcore_barrier()

    def g_start(c, stage, sem):
        pltpu.async_copy(slab.at[idx_v.at[pl.ds(c * CHUNK, CHUNK)]], stage, sem)

    def g_wait(c, stage, sem):
        pltpu.make_async_copy(
            slab.at[idx_v.at[pl.ds(c * CHUNK, CHUNK)]], stage, sem
        ).wait()

    def w_start(c, stage, sem):
        pltpu.async_copy(stage, out_hbm.at[pl.ds(row_base + c * CHUNK, CHUNK)], sem)

    def w_drain(c, stage, sem):
        pltpu.make_async_copy(
            stage, out_hbm.at[pl.ds(row_base + c * CHUNK, CHUNK)], sem
        ).wait()

    g_start(0, stages[0], gsems[0])
    g_start(1, stages[1], gsems[1])

    def ring_step(h, carry):
        for k in range(4):
            c = h * 4 + k
            kn = (k + 2) % 4

            @pl.when(c < NCH)
            def _(c=c, k=k, kn=kn):
                @pl.when(c >= 2)
                def _():
                    w_drain(c - 2, stages[kn], wsems[kn])

                @pl.when(c + 2 < NCH)
                def _():
                    g_start(c + 2, stages[kn], gsems[kn])

                g_wait(c, stages[k], gsems[k])
                w_start(c, stages[k], wsems[k])

        return carry

    lax.fori_loop(0, (NCH + 3) // 4, ring_step, 0)

    w_drain(NCH - 2, stages[(NCH - 2) % 4], wsems[(NCH - 2) % 4])
    w_drain(NCH - 1, stages[(NCH - 1) % 4], wsems[(NCH - 1) % 4])


def kernel(edge_embedding, nbr_idx):
    table = edge_embedding.reshape(NT, F)
    nbr_flat = nbr_idx.reshape(NT)
    out = _gather_kernel(table, nbr_flat)
    return out.reshape(B, AT, K, NBR, F).transpose(0, 1, 3, 2, 4)


# speedup vs baseline: 1.0766x; 1.0766x over previous
import functools

import jax
import jax.numpy as jnp
from jax import lax
from jax.experimental import pallas as pl
from jax.experimental.pallas import tpu as pltpu
from jax.experimental.pallas import tpu_sc as plsc

B, AT, NBR, F = 8, 96, 16, 128
K = NBR - 1
NT = B * AT * NBR
NOUT = NT * K
NW = 32
ROWS_W = NOUT // NW
ATOMS_W = ROWS_W // (K * NBR)
CHUNK = 128
NCH = ROWS_W // CHUNK


@functools.partial(
    pl.kernel,
    mesh=plsc.VectorSubcoreMesh(core_axis_name="c", subcore_axis_name="s"),
    out_type=jax.ShapeDtypeStruct((NOUT, F), jnp.float32),
    compiler_params=pltpu.CompilerParams(needs_layout_passes=False),
    scratch_types=[
        pltpu.VMEM_SHARED((NT // 2, F), jnp.float32),
        pltpu.VMEM((ATOMS_W * NBR,), jnp.int32),
        pltpu.VMEM((ROWS_W,), jnp.int32),
        pltpu.VMEM((CHUNK, F), jnp.float32),
        pltpu.VMEM((CHUNK, F), jnp.float32),
        pltpu.VMEM((CHUNK, F), jnp.float32),
        pltpu.VMEM((CHUNK, F), jnp.float32),
        pltpu.SemaphoreType.DMA,
        pltpu.SemaphoreType.DMA,
        pltpu.SemaphoreType.DMA,
        pltpu.SemaphoreType.DMA,
        pltpu.SemaphoreType.DMA,
        pltpu.SemaphoreType.DMA,
        pltpu.SemaphoreType.DMA,
        pltpu.SemaphoreType.DMA,
        pltpu.SemaphoreType.DMA,
    ],
)
def _gather_kernel(table_hbm, nbr_hbm, out_hbm, slab, nbr_v, idx_v,
                   stage_0, stage_1, stage_2, stage_3,
                   gsem_0, gsem_1, gsem_2, gsem_3,
                   wsem_0, wsem_1, wsem_2, wsem_3, ssem):
    stages = [stage_0, stage_1, stage_2, stage_3]
    gsems = [gsem_0, gsem_1, gsem_2, gsem_3]
    wsems = [wsem_0, wsem_1, wsem_2, wsem_3]
    cid = lax.axis_index("c")
    sid = lax.axis_index("s")
    wid = cid * 16 + sid
    atom_base = wid * ATOMS_W
    row_base = wid * ROWS_W
    mol = atom_base // AT

    slab_rows = NT // 2 // 16
    slab_src = table_hbm.at[pl.ds(cid * (NT // 2) + sid * slab_rows, slab_rows)]
    slab_dst = slab.at[pl.ds(sid * slab_rows, slab_rows)]
    pltpu.async_copy(slab_src, slab_dst, ssem)
    pltpu.sync_copy(nbr_hbm.at[pl.ds(atom_base * NBR, ATOMS_W * NBR)], nbr_v)

    iota = lax.iota(jnp.int32, 16)
    kcols = [s - lax.shift_right_arithmetic(iota - (s + 1), 31) for s in range(K)]

    mol_loc = mol - cid * (B // 2)

    def build_atom(m, carry):
        base_vec = (nbr_v[pl.ds(m * NBR, NBR)] + mol_loc * AT) * NBR
        for s in range(K):
            idx_v[pl.ds((m * K + s) * NBR, NBR)] = base_vec + kcols[s]
        return carry

    lax.fori_loop(0, ATOMS_W, build_atom, 0)

    pltpu.make_async_copy(slab_src, slab_dst, ssem).wait()
    plsc.subcore_barrier()

    def g_start(c, stage, sem):
        pltpu.async_copy(slab.at[idx_v.at[pl.ds(c * CHUNK, CHUNK)]], stage, sem)

    def g_wait(c, stage, sem):
        pltpu.make_async_copy(
            slab.at[idx_v.at[pl.ds(c * CHUNK, CHUNK)]], stage, sem
        ).wait()

    def w_start(c, stage, sem):
        pltpu.async_copy(stage, out_hbm.at[pl.ds(row_base + c * CHUNK, CHUNK)], sem)

    def w_drain(c, stage, sem):
        pltpu.make_async_copy(
            stage, out_hbm.at[pl.ds(row_base + c * CHUNK, CHUNK)], sem
        ).wait()

    g_start(0, stages[0], gsems[0])
    g_start(1, stages[1], gsems[1])

    def ring_step(h, carry):
        for k in range(4):
            c = h * 4 + k
            kn = (k + 2) % 4

            @pl.when(c < NCH)
            def _(c=c, k=k, kn=kn):
                @pl.when(c >= 2)
                def _():
                    w_drain(c - 2, stages[kn], wsems[kn])

                @pl.when(c + 2 < NCH)
                def _():
                    g_start(c + 2, stages[kn], gsems[kn])

                g_wait(c, stages[k], gsems[k])
                w_start(c, stages[k], wsems[k])

        return carry

    lax.fori_loop(0, (NCH + 3) // 4, ring_step, 0)

    w_drain(NCH - 2, stages[(NCH - 2) % 4], wsems[(NCH - 2) % 4])
    w_drain(NCH - 1, stages[(NCH - 1) % 4], wsems[(NCH - 1) % 4])


def kernel(edge_embedding, nbr_idx):
    table = edge_embedding.reshape(NT, F)
    nbr_flat = nbr_idx.reshape(NT)
    out = _gather_kernel(table, nbr_flat)
    return out.reshape(B, AT, K, NBR, F).transpose(0, 1, 3, 2, 4)
